# baseline (speedup 1.0000x reference)
import jax
import jax.numpy as jnp
from jax import lax
from jax.experimental import pallas as pl
from jax.experimental.pallas import tpu as pltpu
from jax.experimental.pallas import tpu_sc as plsc

N_FACTORS = 64

NC = 2
NS = 16
NW = NC * NS

B = 4096 * 200
B_PER_W = B // NW
CHUNK = 128
NCHUNK = B_PER_W // CHUNK
NBUF = 2


def _gather_body(idx_hbm, table_hbm, out_hbm, idx_v, rows_v, gsem, ssem):
    wid = lax.axis_index("s") * NC + lax.axis_index("c")
    base = wid * B_PER_W
    pltpu.sync_copy(idx_hbm.at[pl.ds(base, B_PER_W)], idx_v)

    def _gather(j, buf):
        return pltpu.make_async_copy(
            table_hbm.at[idx_v.at[pl.ds(j * CHUNK, CHUNK)]],
            rows_v.at[buf],
            gsem.at[buf],
        )

    def _store(j, buf):
        return pltpu.make_async_copy(
            rows_v.at[buf],
            out_hbm.at[pl.ds(base + j * CHUNK, CHUNK)],
            ssem.at[buf],
        )

    _gather(0, 0).start()

    @pl.loop(0, NCHUNK - 1)
    def _(j):
        buf = lax.rem(j, NBUF)
        nbuf = lax.rem(j + 1, NBUF)
        _gather(j + 1, nbuf).start()
        _gather(j, buf).wait()
        _store(j, buf).start()
        _store(j, buf).wait()

    last = NCHUNK - 1
    lbuf = last % NBUF
    _gather(last, lbuf).wait()
    _store(last, lbuf).start()
    _store(last, lbuf).wait()


def _sc_gather(idx_flat, table):
    run = pl.kernel(
        _gather_body,
        out_type=jax.ShapeDtypeStruct((B, N_FACTORS), jnp.float32),
        mesh=plsc.VectorSubcoreMesh(core_axis_name="c", subcore_axis_name="s"),
        scratch_types=[
            pltpu.VMEM((B_PER_W,), jnp.int32),
            pltpu.VMEM((NBUF, CHUNK, N_FACTORS), jnp.float32),
            pltpu.SemaphoreType.DMA((NBUF,)),
            pltpu.SemaphoreType.DMA((NBUF,)),
        ],
        compiler_params=pltpu.CompilerParams(use_tc_tiling_on_sc=False),
    )
    return run(idx_flat, table)


def kernel(items, table):
    idx_flat = items.reshape(-1).astype(jnp.int32)
    out = _sc_gather(idx_flat, table)
    return out.reshape(items.shape[0], items.shape[1], N_FACTORS)
